# baseline (speedup 1.0000x reference)
import jax
import jax.numpy as jnp
from jax import lax
from jax.experimental import pallas as pl
from jax.experimental.pallas import tpu as pltpu
from jax.experimental.pallas import tpu_sc as plsc

B = 16384
H = 64
V = 100

_info = plsc.get_sparse_core_info()
_NC, _NS, _L = _info.num_cores, _info.num_subcores, _info.num_lanes
_NW = _NC * _NS
_BPW = B // _NW
_CH = 128
_NCHUNK = _BPW // _CH
_HC = H // _L


def _sc_kernel(z_hbm, z1_hbm, z2_hbm, z3_hbm, table_hbm,
               w1_hbm, b1_hbm, w2_hbm, b2_hbm, w3_hbm, b3_hbm,
               o0_hbm, o1_hbm, o2_hbm, o3_hbm,
               table_v, idx_v, z1_v, z2_v, z3_v,
               o0_v, o1_v, o2_v, o3_v, wb_v, bc_v, sem, wsem):
    wid = lax.axis_index("s") * _NC + lax.axis_index("c")
    base_w = wid * _BPW

    wd = pltpu.async_copy(w1_hbm, wb_v.at[0], wsem)
    for src, j in ((b1_hbm, 1), (w2_hbm, 2), (b2_hbm, 3),
                   (w3_hbm, 4), (b3_hbm, 5)):
        pltpu.async_copy(src, wb_v.at[j], wsem)
    stage = [
        pltpu.async_copy(table_hbm, table_v, sem),
        pltpu.async_copy(z_hbm.at[pl.ds(base_w, _BPW)], idx_v, sem),
        pltpu.async_copy(z1_hbm.at[pl.ds(base_w, _BPW)], z1_v, sem),
        pltpu.async_copy(z2_hbm.at[pl.ds(base_w, _BPW)], z2_v, sem),
        pltpu.async_copy(z3_hbm.at[pl.ds(base_w, _BPW)], z3_v, sem),
    ]
    for _ in range(6):
        wd.wait()

    for c in range(_HC):
        srcs = [wb_v[j, pl.ds(c * _L, _L)] for j in range(6)]
        for lane in range(_L):
            sel = jnp.full((_L,), lane, dtype=jnp.int32)
            off = (c * _L + lane) * _L
            for j in range(6):
                bc_v[j, pl.ds(off, _L)] = srcs[j].at[sel].get(
                    mode="promise_in_bounds")

    outs = (o0_v, o1_v, o2_v, o3_v)
    pend = [[], []]
    for k in range(_NCHUNK):
        p = k % 2
        for d in pend[p]:
            d.wait()
        pend[p] = []
        col0 = k * _CH

        def body(h, col0=col0, p=p):
            hoff = h * _L
            wb1 = bc_v[0, pl.ds(hoff, _L)]
            bb1 = bc_v[1, pl.ds(hoff, _L)]
            wb2 = bc_v[2, pl.ds(hoff, _L)]
            bb2 = bc_v[3, pl.ds(hoff, _L)]
            wb3 = bc_v[4, pl.ds(hoff, _L)]
            bb3 = bc_v[5, pl.ds(hoff, _L)]
            hvv = jnp.full((_L,), h * V, dtype=jnp.int32)
            for g in range(_CH // _L):
                c0 = col0 + g * _L
                gl = g * _L
                gidx = idx_v[pl.ds(c0, _L)] + hvv
                o0_v[p, h, pl.ds(gl, _L)] = plsc.load_gather(
                    table_v, [gidx])
                zv1 = z1_v[pl.ds(c0, _L)]
                o1_v[p, h, pl.ds(gl, _L)] = wb1 * zv1 + bb1
                zv2 = z2_v[pl.ds(c0, _L)]
                o2_v[p, h, pl.ds(gl, _L)] = wb2 * zv2 + bb2
                zv3 = z3_v[pl.ds(c0, _L)]
                o3_v[p, h, pl.ds(gl, _L)] = wb3 * zv3 + bb3

        plsc.parallel_loop(0, H, unroll=2)(body)

        cb = base_w + col0
        pend[p] = [
            pltpu.async_copy(o.at[p], hbm.at[:, pl.ds(cb, _CH)], sem)
            for o, hbm in zip(outs, (o0_hbm, o1_hbm, o2_hbm, o3_hbm))
        ]
    for pp in pend:
        for d in pp:
            d.wait()


def kernel(z, z1, z2, z3, emb_table, W1, b1, W2, b2, W3, b3):
    mesh = plsc.VectorSubcoreMesh(core_axis_name="c", subcore_axis_name="s")
    f32 = jnp.float32
    run = pl.kernel(
        _sc_kernel, mesh=mesh,
        out_type=(
            jax.ShapeDtypeStruct((H, B), f32),
            jax.ShapeDtypeStruct((H, B), f32),
            jax.ShapeDtypeStruct((H, B), f32),
            jax.ShapeDtypeStruct((H, B), f32),
        ),
        scratch_types=[
            pltpu.VMEM((V * H,), f32),
            pltpu.VMEM((_BPW,), jnp.int32),
            pltpu.VMEM((_BPW,), f32),
            pltpu.VMEM((_BPW,), f32),
            pltpu.VMEM((_BPW,), f32),
            pltpu.VMEM((2, H, _CH), f32),
            pltpu.VMEM((2, H, _CH), f32),
            pltpu.VMEM((2, H, _CH), f32),
            pltpu.VMEM((2, H, _CH), f32),
            pltpu.VMEM((6, H), f32),
            pltpu.VMEM((6, H * _L), f32),
            pltpu.SemaphoreType.DMA,
            pltpu.SemaphoreType.DMA,
        ],
        compiler_params=pltpu.CompilerParams(needs_layout_passes=False),
    )
    o0, o1, o2, o3 = run(
        z.astype(jnp.int32), z1.reshape(-1), z2.reshape(-1),
        z3.reshape(-1), emb_table.T.reshape(-1),
        W1.reshape(-1), b1, W2.reshape(-1), b2, W3.reshape(-1), b3)
    return (o0.T, o1.T, o2.T, o3.T)


# speedup vs baseline: 1.0144x; 1.0144x over previous
import jax
import jax.numpy as jnp
from jax import lax
from jax.experimental import pallas as pl
from jax.experimental.pallas import tpu as pltpu
from jax.experimental.pallas import tpu_sc as plsc

B = 16384
H = 64
V = 100

_info = plsc.get_sparse_core_info()
_NC, _NS, _L = _info.num_cores, _info.num_subcores, _info.num_lanes
_NW = _NC * _NS
_BPW = B // _NW
_CH = 128
_NCHUNK = _BPW // _CH
_HC = H // _L


def _sc_kernel(z_hbm, z1_hbm, z2_hbm, z3_hbm, table_hbm,
               w1_hbm, b1_hbm, w2_hbm, b2_hbm, w3_hbm, b3_hbm,
               o0_hbm, o1_hbm, o2_hbm, o3_hbm,
               table_v, idx_v, z1_v, z2_v, z3_v,
               o0_v, o1_v, o2_v, o3_v, wb_v, bc_v, sem, wsem):
    wid = lax.axis_index("s") * _NC + lax.axis_index("c")
    base_w = wid * _BPW

    wd = pltpu.async_copy(w1_hbm, wb_v.at[0], wsem)
    for src, j in ((b1_hbm, 1), (w2_hbm, 2), (b2_hbm, 3),
                   (w3_hbm, 4), (b3_hbm, 5)):
        pltpu.async_copy(src, wb_v.at[j], wsem)
    stage = [
        pltpu.async_copy(table_hbm, table_v, sem),
        pltpu.async_copy(z_hbm.at[pl.ds(base_w, _BPW)], idx_v, sem),
        pltpu.async_copy(z1_hbm.at[pl.ds(base_w, _BPW)], z1_v, sem),
        pltpu.async_copy(z2_hbm.at[pl.ds(base_w, _BPW)], z2_v, sem),
        pltpu.async_copy(z3_hbm.at[pl.ds(base_w, _BPW)], z3_v, sem),
    ]
    for _ in range(6):
        wd.wait()

    for c in range(_HC):
        srcs = [wb_v[j, pl.ds(c * _L, _L)] for j in range(6)]
        for lane in range(_L):
            sel = jnp.full((_L,), lane, dtype=jnp.int32)
            off = (c * _L + lane) * _L
            for j in range(6):
                bc_v[j, pl.ds(off, _L)] = srcs[j].at[sel].get(
                    mode="promise_in_bounds")

    outs = (o0_v, o1_v, o2_v, o3_v)
    pend = [[], []]
    for k in range(_NCHUNK):
        p = k % 2
        for d in pend[p]:
            d.wait()
        pend[p] = []
        col0 = k * _CH

        def body(h, col0=col0, p=p):
            hoff = h * _L
            wb1 = bc_v[0, pl.ds(hoff, _L)]
            bb1 = bc_v[1, pl.ds(hoff, _L)]
            wb2 = bc_v[2, pl.ds(hoff, _L)]
            bb2 = bc_v[3, pl.ds(hoff, _L)]
            wb3 = bc_v[4, pl.ds(hoff, _L)]
            bb3 = bc_v[5, pl.ds(hoff, _L)]
            hvv = jnp.full((_L,), h * V, dtype=jnp.int32)
            for g in range(_CH // _L):
                c0 = col0 + g * _L
                gl = g * _L
                gidx = idx_v[pl.ds(c0, _L)] + hvv
                o0_v[p, h, pl.ds(gl, _L)] = plsc.load_gather(
                    table_v, [gidx])
                zv1 = z1_v[pl.ds(c0, _L)]
                o1_v[p, h, pl.ds(gl, _L)] = wb1 * zv1 + bb1
                zv2 = z2_v[pl.ds(c0, _L)]
                o2_v[p, h, pl.ds(gl, _L)] = wb2 * zv2 + bb2
                zv3 = z3_v[pl.ds(c0, _L)]
                o3_v[p, h, pl.ds(gl, _L)] = wb3 * zv3 + bb3

        plsc.parallel_loop(0, H, unroll=4)(body)

        cb = base_w + col0
        pend[p] = [
            pltpu.async_copy(o.at[p], hbm.at[:, pl.ds(cb, _CH)], sem)
            for o, hbm in zip(outs, (o0_hbm, o1_hbm, o2_hbm, o3_hbm))
        ]
    for pp in pend:
        for d in pp:
            d.wait()


def kernel(z, z1, z2, z3, emb_table, W1, b1, W2, b2, W3, b3):
    mesh = plsc.VectorSubcoreMesh(core_axis_name="c", subcore_axis_name="s")
    f32 = jnp.float32
    run = pl.kernel(
        _sc_kernel, mesh=mesh,
        out_type=(
            jax.ShapeDtypeStruct((H, B), f32),
            jax.ShapeDtypeStruct((H, B), f32),
            jax.ShapeDtypeStruct((H, B), f32),
            jax.ShapeDtypeStruct((H, B), f32),
        ),
        scratch_types=[
            pltpu.VMEM((V * H,), f32),
            pltpu.VMEM((_BPW,), jnp.int32),
            pltpu.VMEM((_BPW,), f32),
            pltpu.VMEM((_BPW,), f32),
            pltpu.VMEM((_BPW,), f32),
            pltpu.VMEM((2, H, _CH), f32),
            pltpu.VMEM((2, H, _CH), f32),
            pltpu.VMEM((2, H, _CH), f32),
            pltpu.VMEM((2, H, _CH), f32),
            pltpu.VMEM((6, H), f32),
            pltpu.VMEM((6, H * _L), f32),
            pltpu.SemaphoreType.DMA,
            pltpu.SemaphoreType.DMA,
        ],
        compiler_params=pltpu.CompilerParams(needs_layout_passes=False),
    )
    o0, o1, o2, o3 = run(
        z.astype(jnp.int32), z1.reshape(-1), z2.reshape(-1),
        z3.reshape(-1), emb_table.T.reshape(-1),
        W1.reshape(-1), b1, W2.reshape(-1), b2, W3.reshape(-1), b3)
    return (o0.T, o1.T, o2.T, o3.T)
